# baseline (speedup 1.0000x reference)
import functools

import jax
import jax.numpy as jnp
from jax import lax
from jax.experimental import pallas as pl
from jax.experimental.pallas import tpu as pltpu
from jax.experimental.pallas import tpu_sc as plsc

NUM_EMB = 1000000
DIM = 64
NI = 16384
NJ = 20
NUM_CORES = 2
NUM_SUBCORES = 16
NW = NUM_CORES * NUM_SUBCORES
BLK = 128
NBI = NI // BLK
NBLK = NJ * NBI
BPW = NBLK // NW
NPAIR = BPW // 2


def _make_embedding_kernel():
    mesh = plsc.VectorSubcoreMesh(core_axis_name="c", subcore_axis_name="s")

    @functools.partial(
        pl.kernel,
        mesh=mesh,
        compiler_params=pltpu.CompilerParams(needs_layout_passes=False),
        out_type=jax.ShapeDtypeStruct((NJ, DIM, NI), jnp.float32),
        scratch_types=[
            pltpu.VMEM((2 * NI,), jnp.int32),
            pltpu.VMEM((BLK,), jnp.int32),
            pltpu.VMEM((BLK,), jnp.int32),
            pltpu.VMEM((BLK,), jnp.int32),
            pltpu.VMEM((BLK,), jnp.int32),
            pltpu.VMEM((BLK, 2 * DIM + 1), jnp.float32),
            pltpu.VMEM((BLK, 2 * DIM + 1), jnp.float32),
            pltpu.VMEM((DIM, BLK), jnp.float32),
            pltpu.VMEM((DIM, BLK), jnp.float32),
            pltpu.SemaphoreType.DMA,
            pltpu.SemaphoreType.DMA,
        ],
    )
    def emb(tok2d, w128, o3, idxb, m0, m1, h0, h1, r0, r1, o0, o1, sem_g, sem_s):
        wid = lax.axis_index("s") * NUM_CORES + lax.axis_index("c")
        base = wid * BPW
        j0 = lax.div(base, NBI)
        j1 = lax.div(base + BPW - 1, NBI)
        pltpu.sync_copy(tok2d.at[j0], idxb.at[pl.ds(0, NI)])
        pltpu.sync_copy(tok2d.at[j1], idxb.at[pl.ds(NI, NI)])

        ms = (m0, m1)
        hs = (h0, h1)
        rs = (r0, r1)
        os_ = (o0, o1)
        iotas = [lax.iota(jnp.int32, 16) + 16 * g for g in range(8)]

        def prep(blk, m_ref, h_ref):
            jj = lax.div(blk, NBI)
            ti = lax.rem(blk, NBI)
            off = (jj - j0) * NI + ti * BLK
            for g in range(8):
                v = idxb[pl.ds(off + 16 * g, 16)]
                m_ref[pl.ds(16 * g, 16)] = lax.shift_right_logical(v, 1)
                h_ref[pl.ds(16 * g, 16)] = lax.shift_left(v & 1, 6)

        def gather_start(m_ref, r_ref):
            pltpu.make_async_copy(
                w128.at[m_ref], r_ref.at[:, pl.ds(0, 2 * DIM)], sem_g
            ).start()

        def gather_wait(r_ref):
            pltpu.make_async_copy(
                w128.at[ms[0]], r_ref.at[:, pl.ds(0, 2 * DIM)], sem_g
            ).wait()

        def extract(r_ref, h_ref, o_ref):
            hvs = [h_ref[pl.ds(16 * g, 16)] for g in range(8)]
            for cc in range(DIM):
                xs = [
                    plsc.load_gather(r_ref, [iotas[g], hvs[g] + cc])
                    for g in range(8)
                ]
                for g in range(8):
                    o_ref[cc, pl.ds(16 * g, 16)] = xs[g]

        def store_start(blk, o_ref):
            jj = lax.div(blk, NBI)
            ti = lax.rem(blk, NBI)
            pltpu.make_async_copy(
                o_ref, o3.at[jj, :, pl.ds(ti * BLK, BLK)], sem_s
            ).start()

        def store_wait(o_ref):
            pltpu.make_async_copy(
                o_ref, o3.at[0, :, pl.ds(0, BLK)], sem_s
            ).wait()

        prep(base, m0, h0)
        gather_start(m0, r0)

        def pair_step(p, carry):
            for b in (0, 1):
                blk = base + 2 * p + b
                nxt = 1 - b

                @pl.when(2 * p + b < BPW - 1)
                def _():
                    prep(blk + 1, ms[nxt], hs[nxt])
                    gather_start(ms[nxt], rs[nxt])

                gather_wait(rs[b])

                @pl.when(2 * p + b >= 2)
                def _():
                    store_wait(os_[b])

                extract(rs[b], hs[b], os_[b])
                store_start(blk, os_[b])
            return carry

        lax.fori_loop(0, NPAIR, pair_step, 0)
        store_wait(o0)
        store_wait(o1)

    return emb


_emb = _make_embedding_kernel()


def kernel(token_ids, weight):
    tok2d = token_ids.T
    w128 = weight.reshape(NUM_EMB // 2, 2 * DIM)
    o3 = _emb(tok2d, w128)
    return o3.transpose(2, 0, 1)


# speedup vs baseline: 1.1063x; 1.1063x over previous
import functools

import jax
import jax.numpy as jnp
from jax import lax
from jax.experimental import pallas as pl
from jax.experimental.pallas import tpu as pltpu
from jax.experimental.pallas import tpu_sc as plsc

NUM_EMB = 1000000
DIM = 64
TOKENS = 16384 * 20
NUM_CORES = 2
NUM_SUBCORES = 16
NW = NUM_CORES * NUM_SUBCORES
ROWS_PER_W = TOKENS // NW
CHUNK = 128
NCHUNK = ROWS_PER_W // CHUNK


def _make_embedding_kernel():
    mesh = plsc.VectorSubcoreMesh(core_axis_name="c", subcore_axis_name="s")

    @functools.partial(
        pl.kernel,
        mesh=mesh,
        compiler_params=pltpu.CompilerParams(use_tc_tiling_on_sc=False),
        out_type=jax.ShapeDtypeStruct((TOKENS, DIM), jnp.float32),
        scratch_types=[
            pltpu.VMEM((ROWS_PER_W,), jnp.int32),
            pltpu.VMEM((2, CHUNK, DIM), jnp.float32),
            pltpu.SemaphoreType.DMA,
            pltpu.SemaphoreType.DMA,
        ],
    )
    def emb(tok_hbm, w_hbm, out_hbm, idx_v, rows_v, sem_g, sem_s):
        wid = lax.axis_index("s") * NUM_CORES + lax.axis_index("c")
        base = wid * ROWS_PER_W
        pltpu.sync_copy(tok_hbm.at[pl.ds(base, ROWS_PER_W)], idx_v)

        def gather_start(j, b):
            pltpu.make_async_copy(
                w_hbm.at[idx_v.at[pl.ds(j * CHUNK, CHUNK)]], rows_v.at[b], sem_g
            ).start()

        def gather_wait(b):
            pltpu.make_async_copy(
                w_hbm.at[idx_v.at[pl.ds(0, CHUNK)]], rows_v.at[b], sem_g
            ).wait()

        def store_start(j, b):
            pltpu.make_async_copy(
                rows_v.at[b], out_hbm.at[pl.ds(base + j * CHUNK, CHUNK)], sem_s
            ).start()

        def store_wait(b):
            pltpu.make_async_copy(
                rows_v.at[b], out_hbm.at[pl.ds(base, CHUNK)], sem_s
            ).wait()

        gather_start(0, 0)

        def step(j, carry):
            b = lax.rem(j, 2)
            gather_wait(b)

            @pl.when(j > 0)
            def _():
                store_wait(1 - b)

            @pl.when(j + 1 < NCHUNK)
            def _():
                gather_start(j + 1, 1 - b)

            store_start(j, b)
            return carry

        lax.fori_loop(0, NCHUNK, step, 0)
        store_wait(lax.rem(NCHUNK - 1, 2))

    return emb


_emb = _make_embedding_kernel()


def kernel(token_ids, weight):
    tok = token_ids.reshape(TOKENS)
    out = _emb(tok, weight)
    return out.reshape(16384, 20, DIM)
